# baseline (speedup 1.0000x reference)
import functools

import numpy as np
import jax
import jax.numpy as jnp
from jax import lax
from jax.experimental import pallas as pl
from jax.experimental.pallas import tpu as pltpu
from jax.experimental.pallas import tpu_sc as plsc

N = 10000
E = 160000
D = 128
H = 8
DP = 16
TEMP = float(np.sqrt(DP))

NW = 32
CHUNK = 128
NCH = E // CHUNK
NPT = 320
NPAD = NW * NPT
OFFS_PAD = NPAD + NPT + 16
TILE_E = 1000
TILE_N = 400
NEG = -3.0e38

_P2 = np.array([d * H + h for h in range(H) for d in range(DP)])


def _prep_weights(p):
    f32 = jnp.float32
    Wq, bq = p['proj_q'][0]['w'], p['proj_q'][0]['b']
    Wk, bk = p['proj_k'][0]['w'], p['proj_k'][0]['b']
    Wv, bv = p['proj_v'][0]['w'], p['proj_v'][0]['b']
    Az = np.zeros((D, 2 * D), np.float32)
    Bz = np.zeros((D, 2 * D), np.float32)
    bz = np.zeros((2 * D,), np.float32)
    Az = jnp.asarray(Az)
    Bz = jnp.asarray(Bz)
    bz = jnp.asarray(bz)
    for h in range(H):
        qcols = np.array([d * H + h for d in range(DP)])
        Az = Az.at[:, h * 32:h * 32 + 16].set(Wq[:, qcols])
        Bz = Bz.at[:, h * 32 + 16:h * 32 + 32].set(Wk[:, qcols])
        bz = bz.at[h * 32:h * 32 + 16].set(bq[qcols])
        bz = bz.at[h * 32 + 16:h * 32 + 32].set(bk[qcols])
    w1, b1 = p['nn_att'][0]['w'], p['nn_att'][0]['b']
    w2, b2 = p['nn_att'][1]['w'], p['nn_att'][1]['b']
    I8 = jnp.eye(H, dtype=f32)
    W1e = jnp.kron(I8, w1)
    W2e = jnp.kron(I8, w2)
    b1e = jnp.tile(b1, H)
    b2e = jnp.tile(b2, H)
    Wn1 = p['nn_node_update'][0]['w']
    d = {
        'We1': p['nn_edge_update'][0]['w'], 'be1': p['nn_edge_update'][0]['b'][None, :],
        'We2': p['nn_edge_update'][1]['w'], 'be2': p['nn_edge_update'][1]['b'][None, :],
        'Az': Az, 'Bz': Bz, 'bz': bz[None, :],
        'W1e': W1e, 'b1e': b1e[None, :], 'W2e': W2e, 'b2e': b2e[None, :],
        'Wvp': Wv[:, _P2], 'bvp': bv[_P2][None, :],
        'Wd1': p['distance_mlp'][0]['w'], 'bd1': p['distance_mlp'][0]['b'][None, :],
        'Wd2': p['distance_mlp'][1]['w'], 'bd2': p['distance_mlp'][1]['b'][None, :],
        'Wn1a': Wn1[:D, :], 'Wn1b': Wn1[D:, :][_P2, :],
        'bn1': p['nn_node_update'][0]['b'][None, :],
        'Wn2': p['nn_node_update'][1]['w'], 'bn2': p['nn_node_update'][1]['b'][None, :],
        'Wa1a': p['edge_attention_mlp'][0]['w'][:D, :],
        'Wa1b': p['edge_attention_mlp'][0]['w'][D:, :],
        'ba1': p['edge_attention_mlp'][0]['b'][None, :],
    }
    return d


EPT = E // NW
NFC = EPT // CHUNK
TAIL = EPT - NFC * CHUNK


def _make_gather():
    mesh = plsc.VectorSubcoreMesh(core_axis_name="c", subcore_axis_name="s")
    out_type = tuple(jax.ShapeDtypeStruct((E, D), jnp.float32)
                     for _ in range(3))
    scratch = []
    for _ in range(3):
        scratch.append(pltpu.VMEM((EPT,), jnp.int32))
        scratch.append(pltpu.VMEM((CHUNK, D), jnp.float32))
        scratch.append(pltpu.VMEM((CHUNK, D), jnp.float32))
        scratch.append(pltpu.SemaphoreType.DMA)
        scratch.append(pltpu.SemaphoreType.DMA)
        scratch.append(pltpu.SemaphoreType.DMA)
        scratch.append(pltpu.SemaphoreType.DMA)

    @functools.partial(pl.kernel, mesh=mesh, out_type=out_type,
                       scratch_types=tuple(scratch), name="sc_gather",
                       compiler_params=pltpu.CompilerParams(
                           needs_layout_passes=False))
    def k(xh, efh, rowh, colh, revh, xr_o, xc_o, rf_o, *scr):
        tables = (xh, xh, efh)
        idxs = (rowh, colh, revh)
        outs = (xr_o, xc_o, rf_o)
        ivs = [scr[7 * g] for g in range(3)]
        bufs = [(scr[7 * g + 1], scr[7 * g + 2]) for g in range(3)]
        gsem = [(scr[7 * g + 3], scr[7 * g + 4]) for g in range(3)]
        osem = [(scr[7 * g + 5], scr[7 * g + 6]) for g in range(3)]
        wid = lax.axis_index("s") * 2 + lax.axis_index("c")
        base = wid * EPT

        def start_gather(g, b, j):
            pltpu.async_copy(
                tables[g].at[ivs[g].at[pl.ds(j * CHUNK, CHUNK)]],
                bufs[g][b], gsem[g][b])

        def wait_gather(g, b):
            pltpu.make_async_copy(
                tables[g].at[ivs[g].at[pl.ds(0, CHUNK)]],
                bufs[g][b], gsem[g][b]).wait()

        def start_out(g, b, j):
            pltpu.async_copy(
                bufs[g][b], outs[g].at[pl.ds(base + j * CHUNK, CHUNK)],
                osem[g][b])

        def wait_out(g, b):
            pltpu.make_async_copy(
                bufs[g][b], outs[g].at[pl.ds(base, CHUNK)],
                osem[g][b]).wait()

        for g in range(3):
            pltpu.sync_copy(idxs[g].at[pl.ds(base, EPT)], ivs[g])
        for b in range(2):
            for g in range(3):
                start_gather(g, b, b)

        def body(jj, carry):
            for b in range(2):
                j = 2 * jj + b

                @pl.when(j < NFC)
                def _():
                    for g in range(3):
                        wait_gather(g, b)
                    for g in range(3):
                        start_out(g, b, j)
                    for g in range(3):
                        wait_out(g, b)

                    @pl.when(j + 2 < NFC)
                    def _():
                        for g in range(3):
                            start_gather(g, b, j + 2)
            return carry

        lax.fori_loop(0, (NFC + 1) // 2, body, 0)

        for g in range(3):
            pltpu.async_copy(
                tables[g].at[ivs[g].at[pl.ds(NFC * CHUNK, TAIL)]],
                bufs[g][0].at[pl.ds(0, TAIL)], gsem[g][0])
        for g in range(3):
            pltpu.make_async_copy(
                tables[g].at[ivs[g].at[pl.ds(0, TAIL)]],
                bufs[g][0].at[pl.ds(0, TAIL)], gsem[g][0]).wait()
            pltpu.sync_copy(bufs[g][0].at[pl.ds(0, TAIL)],
                            outs[g].at[pl.ds(base + NFC * CHUNK, TAIL)])

    return k


def _make_dd():
    mesh = plsc.VectorSubcoreMesh(core_axis_name="c", subcore_axis_name="s")
    scratch = (
        pltpu.VMEM((N * 4,), jnp.float32),
        pltpu.VMEM((EPT + 16,), jnp.int32),
        pltpu.VMEM((EPT + 16,), jnp.int32),
        pltpu.VMEM((CHUNK * 8,), jnp.float32),
    )

    @functools.partial(
        pl.kernel, mesh=mesh,
        out_type=jax.ShapeDtypeStruct((E * 8,), jnp.float32),
        scratch_types=scratch, name="sc_dd",
        compiler_params=pltpu.CompilerParams(needs_layout_passes=False))
    def k(posh, rowh, colh, dd_o, pos_v, ivr, ivc, dd_v):
        wid = lax.axis_index("s") * 2 + lax.axis_index("c")
        base = wid * EPT
        pltpu.sync_copy(posh, pos_v)
        pltpu.sync_copy(rowh.at[pl.ds(base, EPT)], ivr.at[pl.ds(0, EPT)])
        pltpu.sync_copy(colh.at[pl.ds(base, EPT)], ivc.at[pl.ds(0, EPT)])
        zi = jnp.zeros((DP,), jnp.int32)
        ivr[pl.ds(EPT, DP)] = zi
        ivc[pl.ds(EPT, DP)] = zi
        ii = lax.iota(jnp.int32, DP)

        def do_group(goff, g):
            rv16 = ivr[pl.ds(goff + g * DP, DP)]
            cv16 = ivc[pl.ds(goff + g * DP, DP)]
            d2 = jnp.zeros((DP,), jnp.float32)
            ridx = (g * DP + ii) * 8
            for cc in range(3):
                pa = plsc.load_gather(pos_v, [rv16 * 4 + cc])
                pb = plsc.load_gather(pos_v, [cv16 * 4 + cc])
                dc = pa - pb
                d2 = d2 + dc * dc
                plsc.store_scatter(dd_v, [ridx + cc], dc)
            plsc.store_scatter(dd_v, [ridx + 3], d2)

        def body(j, carry):
            for g in range(8):
                do_group(j * CHUNK, g)
            pltpu.sync_copy(
                dd_v, dd_o.at[pl.ds((base + j * CHUNK) * 8, CHUNK * 8)])
            return carry

        lax.fori_loop(0, NFC, body, 0)
        do_group(NFC * CHUNK, 0)
        pltpu.sync_copy(dd_v.at[pl.ds(0, TAIL * 8)],
                        dd_o.at[pl.ds((base + NFC * CHUNK) * 8, TAIL * 8)])

    return k


CH2 = 64
NCH2 = E // CH2
NSUMPAD = 10048
NZCH = NSUMPAD // CH2


def _make_reduce():
    mesh = plsc.VectorSubcoreMesh(core_axis_name="c", subcore_axis_name="s")
    out_type = (
        jax.ShapeDtypeStruct((NPAD, D), jnp.float32),
        jax.ShapeDtypeStruct((NSUMPAD, D), jnp.float32),
        jax.ShapeDtypeStruct((NSUMPAD, D), jnp.float32),
    )
    scratch = (
        pltpu.VMEM((NPT, D), jnp.float32),
        pltpu.VMEM((CH2, D), jnp.float32),
        pltpu.VMEM((CH2,), jnp.int32),
        pltpu.VMEM((NPT + 24,), jnp.int32),
        pltpu.VMEM((CH2 + 16,), jnp.int32),
        pltpu.VMEM_SHARED((NSUMPAD, D), jnp.float32),
        pltpu.SemaphoreType.DMA,
    )

    @functools.partial(pl.kernel, mesh=mesh, out_type=out_type,
                       scratch_types=scratch, name="sc_reduce",
                       compiler_params=pltpu.CompilerParams(
                           needs_layout_passes=False))
    def k(ue, wv, rowh, colh, perm, srowp, offs,
          agg, osum, isum,
          slab, rows, iv, offs_s, srow_s, shsum, sem):
        cidx = lax.axis_index("c")
        sid = lax.axis_index("s")
        wid = sid * 2 + cidx

        zv = jnp.zeros((DP,), jnp.float32)

        def zb(i, c):
            for kk in range(8):
                rows[i, pl.ds(kk * DP, DP)] = zv
            return c
        lax.fori_loop(0, CH2, zb, 0)

        def zcopy(t, c):
            zc = sid + 16 * t

            @pl.when(zc < NZCH)
            def _():
                pltpu.sync_copy(rows, shsum.at[pl.ds(zc * CH2, CH2)])
            return c
        lax.fori_loop(0, (NZCH + 15) // 16, zcopy, 0)
        plsc.subcore_barrier()

        nlo = wid * NPT
        pltpu.sync_copy(offs.at[pl.ds(nlo, NPT + 8)],
                        offs_s.at[pl.ds(0, NPT + 8)])
        elo = offs_s[pl.ds(0, DP)][0]
        ehi = offs_s[pl.ds(NPT, DP)][0]
        neg = jnp.full((DP,), NEG, jnp.float32)

        def initb(i, c):
            for kk in range(8):
                slab[i, pl.ds(kk * DP, DP)] = neg
            return c
        lax.fori_loop(0, NPT, initb, 0)

        start = (elo // CH2) * CH2
        nch = (ehi - start + CH2 - 1) // CH2

        def chunk_body(j, c):
            off = start + j * CH2
            pltpu.sync_copy(perm.at[pl.ds(off, CH2)], iv)
            pltpu.async_copy(wv.at[iv], rows, sem).wait()
            pltpu.sync_copy(srowp.at[pl.ds(off, CH2)],
                            srow_s.at[pl.ds(0, CH2)])

            def edge_body(i, c2):
                p = off + i

                @pl.when((p >= elo) & (p < ehi))
                def _():
                    ln = srow_s[pl.ds(i, DP)][0] - nlo
                    for kk in range(8):
                        cur = slab[ln, pl.ds(kk * DP, DP)]
                        val = rows[i, pl.ds(kk * DP, DP)]
                        slab[ln, pl.ds(kk * DP, DP)] = jnp.maximum(cur, val)
                return c2
            lax.fori_loop(0, CH2, edge_body, 0)
            return c
        lax.fori_loop(0, nch, chunk_body, 0)
        pltpu.sync_copy(slab, agg.at[pl.ds(nlo, NPT)])

        def sum_body(t, c):
            cid = sid + 16 * t

            @pl.when(cid < NCH2)
            def _():
                @pl.when(cidx == 0)
                def _():
                    pltpu.sync_copy(rowh.at[pl.ds(cid * CH2, CH2)], iv)

                @pl.when(cidx == 1)
                def _():
                    pltpu.sync_copy(colh.at[pl.ds(cid * CH2, CH2)], iv)
                pltpu.sync_copy(ue.at[pl.ds(cid * CH2, CH2)], rows)
                pltpu.sync_copy(rows, shsum.at[iv], add=True)
            return c
        lax.fori_loop(0, (NCH2 + 15) // 16, sum_body, 0)
        plsc.subcore_barrier()

        def wb(t, c):
            zc = sid + 16 * t

            @pl.when(zc < NZCH)
            def _():
                @pl.when(cidx == 0)
                def _():
                    pltpu.sync_copy(shsum.at[pl.ds(zc * CH2, CH2)],
                                    osum.at[pl.ds(zc * CH2, CH2)])

                @pl.when(cidx == 1)
                def _():
                    pltpu.sync_copy(shsum.at[pl.ds(zc * CH2, CH2)],
                                    isum.at[pl.ds(zc * CH2, CH2)])
            return c
        lax.fori_loop(0, (NZCH + 15) // 16, wb, 0)

    return k


def _edge_body(first, xr_ref, xc_ref, rf_ref, ef_ref, aux1_ref, aux2_ref,
               We1, be1, We2, be2, Az, Bz, bz, W1e, b1e, W2e, b2e,
               Wvp, bvp, Wd1, bd1, Wd2, bd2, *out_refs):
    if first:
        ue_ref, wv_ref, dd_ref = out_refs
        dds = aux1_ref[...]
        fm = aux2_ref[:, 0:1]
        dif = dds[:, :3]
        dist = jnp.sqrt(dds[:, 3:4])
        dd = jnp.concatenate([dif, dist], axis=1)
        dd_ref[...] = jnp.concatenate(
            [dif, dist, fm, jnp.zeros((TILE_E, 3), jnp.float32)], axis=1)
    else:
        ue_ref, wv_ref = out_refs
        dd8 = aux1_ref[...]
        fm = dd8[:, 4:5]
        dd = dd8[:, :4]
    xr = xr_ref[...]
    xc = xc_ref[...]
    ef = ef_ref[...]
    refm = rf_ref[...] * fm

    dm = jax.nn.sigmoid(
        jnp.dot(jax.nn.relu(jnp.dot(dd, Wd1[...]) + bd1[...]), Wd2[...])
        + bd2[...])

    h1 = jax.nn.relu(
        jnp.dot(xr, We1[:D, :]) + jnp.dot(ef, We1[D:2 * D, :])
        + jnp.dot(refm, We1[2 * D:3 * D, :]) + jnp.dot(xc, We1[3 * D:, :])
        + be1[...])
    ue_ref[...] = jnp.dot(h1, We2[...]) + be2[...]

    z = jnp.dot(xr, Az[...]) + jnp.dot(ef, Bz[...]) + bz[...]
    att = jnp.dot(jax.nn.relu(jnp.dot(z, W1e[...]) + b1e[...]), W2e[...]) \
        + b2e[...]
    att = att * dm * (1.0 / TEMP)
    a3 = att.reshape(TILE_E, H, DP)
    m = jnp.max(a3, axis=2, keepdims=True)
    ex = jnp.exp(a3 - m)
    prob = (ex / jnp.sum(ex, axis=2, keepdims=True)).reshape(TILE_E, D)
    v = jnp.dot(xc, Wvp[...]) + bvp[...]
    wv_ref[...] = prob * v


_W_EDGE = ['We1', 'be1', 'We2', 'be2', 'Az', 'Bz', 'bz', 'W1e', 'b1e',
           'W2e', 'b2e', 'Wvp', 'bvp', 'Wd1', 'bd1', 'Wd2', 'bd2']
_W_NODE = ['Wn1a', 'Wn1b', 'bn1', 'Wn2', 'bn2', 'Wa1a', 'Wa1b', 'ba1']


def _full_spec(a):
    nd = a.ndim
    return pl.BlockSpec(a.shape, lambda i: (0,) * nd)


def _edge_call(first, xr, xc, rf, ef, aux1, aux2, w):
    ge = E // TILE_E
    row_spec = pl.BlockSpec((TILE_E, D), lambda i: (i, 0))
    aux_spec = lambda a: pl.BlockSpec((TILE_E, a.shape[1]), lambda i: (i, 0))
    weights = [w[k] for k in _W_EDGE]
    in_specs = ([row_spec] * 4 + [aux_spec(aux1), aux_spec(aux2)]
                + [_full_spec(a) for a in weights])
    out_shape = [jax.ShapeDtypeStruct((E, D), jnp.float32),
                 jax.ShapeDtypeStruct((E, D), jnp.float32)]
    out_specs = [row_spec, row_spec]
    if first:
        out_shape.append(jax.ShapeDtypeStruct((E, 8), jnp.float32))
        out_specs.append(pl.BlockSpec((TILE_E, 8), lambda i: (i, 0)))
    return pl.pallas_call(
        functools.partial(_edge_body, first),
        name="tc_edge",
        grid=(ge,),
        in_specs=in_specs,
        out_specs=out_specs,
        out_shape=out_shape,
    )(xr, xc, rf, ef, aux1, aux2, *weights)


def _node_body(x_ref, agg_ref, os_ref, is_ref, cn_ref,
               Wn1a, Wn1b, bn1, Wn2, bn2, Wa1a, Wa1b, ba1, out_ref):
    x = x_ref[...]
    cn = cn_ref[...]
    co = cn[:, 0:1]
    ci = cn[:, 1:2]
    agg = jnp.where(co > 0, agg_ref[...], 0.0)
    un = jnp.dot(
        jax.nn.relu(jnp.dot(x, Wn1a[...]) + jnp.dot(agg, Wn1b[...])
                    + bn1[...]),
        Wn2[...]) + bn2[...]
    om = os_ref[...] / jnp.maximum(co, 1.0)
    im = is_ref[...] / jnp.maximum(ci, 1.0)
    ea = jax.nn.sigmoid(jnp.dot(om, Wa1a[...]) + jnp.dot(im, Wa1b[...])
                        + ba1[...])
    out_ref[...] = jax.nn.relu(un) * ea


def _node_call(x, agg, osum, isum, cnts, w):
    gn = N // TILE_N
    row_spec = pl.BlockSpec((TILE_N, D), lambda i: (i, 0))
    weights = [w[k] for k in _W_NODE]
    in_specs = [row_spec, row_spec, row_spec, row_spec,
                pl.BlockSpec((TILE_N, 8), lambda i: (i, 0))] \
        + [_full_spec(a) for a in weights]
    return pl.pallas_call(
        _node_body,
        name="tc_node",
        grid=(gn,),
        in_specs=in_specs,
        out_specs=row_spec,
        out_shape=jax.ShapeDtypeStruct((N, D), jnp.float32),
    )(x, agg, osum, isum, cnts, *weights)


def kernel(x, edge_feature, edge_index, node_positions, params):
    row = edge_index[0]
    col = edge_index[1]

    keys = row * N + col
    order = jnp.argsort(keys)
    skeys = keys[order]
    rev = col * N + row
    pos = jnp.clip(jnp.searchsorted(skeys, rev), 0, E - 1)
    found = skeys[pos] == rev
    rev_idx = jnp.where(found, order[pos], 0).astype(jnp.int32)
    fmask = found.astype(jnp.float32)

    perm = order.astype(jnp.int32)
    srowp = (skeys // N).astype(jnp.int32)
    offs = jnp.searchsorted(srowp, jnp.arange(OFFS_PAD, dtype=jnp.int32)
                            ).astype(jnp.int32)
    cnt_out = (offs[1:N + 1] - offs[:N]).astype(jnp.float32)
    cnt_in = jnp.bincount(col, length=N).astype(jnp.float32)
    cnts = jnp.zeros((N, 8), jnp.float32)
    cnts = cnts.at[:, 0].set(cnt_out).at[:, 1].set(cnt_in)

    pos4 = jnp.zeros((N, 4), jnp.float32).at[:, :3].set(node_positions)
    fm8 = jnp.zeros((E, 8), jnp.float32).at[:, 0].set(fmask)

    wts = [_prep_weights(params['layer%d' % l]) for l in range(2)]

    gather_k = _make_gather()
    dd_k = _make_dd()
    reduce_k = _make_reduce()

    ef = edge_feature
    for l in range(2):
        w = wts[l]
        if l == 0:
            dds = dd_k(pos4.reshape(-1), row, col)
            xr, xc, rf = gather_k(x, ef, row, col, rev_idx)
            ue, wv, dd8 = _edge_call(True, xr, xc, rf, ef,
                                     dds.reshape(E, 8), fm8, w)
        else:
            xr, xc, rf = gather_k(x, ef, row, col, rev_idx)
            ue, wv = _edge_call(False, xr, xc, rf, ef, dd8, dd8, w)
        agg, osum, isum = reduce_k(ue, wv, row, col, perm, srowp, offs)
        x = _node_call(x, agg, osum, isum, cnts, w)
        ef = ue
    return x
